# baseline (speedup 1.0000x reference)
import functools

import jax
import jax.numpy as jnp
from jax import lax
from jax.experimental import pallas as pl
from jax.experimental.pallas import tpu as pltpu
from jax.experimental.pallas import tpu_sc as plsc

_NC = 2
_NS = 16
_NW = _NC * _NS


def _build(b, s, d, dtype):
    b_per_w = b // _NW
    nbuf = 4

    mesh = plsc.VectorSubcoreMesh(core_axis_name="c", subcore_axis_name="s")

    @functools.partial(
        pl.kernel,
        out_type=jax.ShapeDtypeStruct((b, s, d), dtype),
        mesh=mesh,
        scratch_types=[
            pltpu.VMEM((b_per_w, s), jnp.int32),
            pltpu.VMEM((nbuf, s, d), dtype),
            [pltpu.SemaphoreType.DMA] * nbuf,
        ],
        compiler_params=pltpu.CompilerParams(use_tc_tiling_on_sc=False),
    )
    def emb(idx_hbm, table_hbm, out_hbm, idx_v, rows_v, gsems):
        wid = lax.axis_index("s") * _NC + lax.axis_index("c")
        base = wid * b_per_w
        pltpu.sync_copy(idx_hbm.at[pl.ds(base, b_per_w)], idx_v)

        for u in range(nbuf):
            pltpu.async_copy(table_hbm.at[idx_v.at[u]], rows_v.at[u], gsems[u])

        @pl.loop(0, b_per_w, step=nbuf)
        def _group(g):
            for u in range(nbuf):
                j = g + u
                pltpu.make_async_copy(
                    table_hbm.at[idx_v.at[j]], rows_v.at[u], gsems[u]
                ).wait()
                pltpu.sync_copy(rows_v.at[u], out_hbm.at[base + j])

                @pl.when(j + nbuf < b_per_w)
                def _prefetch():
                    pltpu.async_copy(
                        table_hbm.at[idx_v.at[j + nbuf]], rows_v.at[u], gsems[u]
                    )

    return emb


def kernel(x, table):
    b, s = x.shape
    v, d = table.shape
    return _build(b, s, d, table.dtype)(x.astype(jnp.int32), table)
